# Initial kernel scaffold; baseline (speedup 1.0000x reference)
import jax
import jax.numpy as jnp
from jax.experimental import pallas as pl


def kernel(ts_seq, year_table, month_table, day_table, W1, b1, W2, b2, gamma, beta):
    raise NotImplementedError("write your pallas kernel here")


# speedup vs baseline: 4.9851x; 4.9851x over previous
import functools

import jax
import jax.numpy as jnp
from jax import lax
from jax.experimental import pallas as pl
from jax.experimental.pallas import tpu as pltpu
from jax.experimental.pallas import tpu_sc as plsc

B, L, H = 4096, 50, 32
MAX_YEARS = 50
BASE_YEAR = 2000
NY, NM, ND = MAX_YEARS + 1, 13, 32
NCOMBO = NY * NM * ND
N_TOK = B * L

NW = 32
TOK_PER_W = N_TOK // NW
GBLK = 128
NBLK = TOK_PER_W // GBLK


def _table_body(yt, mt, dt, w1, b1, w2, b2, gamma, beta, t_out):
    f32 = jnp.float32
    yp = jnp.dot(yt[...], w1[0:H, :], preferred_element_type=f32)
    mp = jnp.dot(mt[...], w1[H:2 * H, :], preferred_element_type=f32)
    dp = jnp.dot(dt[...], w1[2 * H:3 * H, :], preferred_element_type=f32)

    r = lax.broadcasted_iota(jnp.int32, (NCOMBO, 1), 0)
    yi = r // (NM * ND)
    mi = (r // ND) % NM
    di = r % ND
    oh_y = (yi == lax.broadcasted_iota(jnp.int32, (NCOMBO, NY), 1)).astype(f32)
    oh_m = (mi == lax.broadcasted_iota(jnp.int32, (NCOMBO, NM), 1)).astype(f32)
    oh_d = (di == lax.broadcasted_iota(jnp.int32, (NCOMBO, ND), 1)).astype(f32)

    s = (jnp.dot(oh_y, yp, preferred_element_type=f32)
         + jnp.dot(oh_m, mp, preferred_element_type=f32)
         + jnp.dot(oh_d, dp, preferred_element_type=f32)
         + b1[...])
    h = jnp.maximum(s, 0.0)
    o = jnp.dot(h, w2[...], preferred_element_type=f32) + b2[...]
    mu = jnp.mean(o, axis=-1, keepdims=True)
    var = jnp.mean(jnp.square(o - mu), axis=-1, keepdims=True)
    normed = (o - mu) * lax.rsqrt(var + 1e-5)
    t_out[...] = normed * gamma[...] + beta[...]


def _build_table(year_table, month_table, day_table, W1, b1, W2, b2, gamma, beta):
    return pl.pallas_call(
        _table_body,
        out_shape=jax.ShapeDtypeStruct((NCOMBO, H), jnp.float32),
    )(year_table, month_table, day_table, W1,
      b1.reshape(1, H), W2, b2.reshape(1, H),
      gamma.reshape(1, H), beta.reshape(1, H))


def _combo_from_ts(t):
    i32 = jnp.int32

    def div(a, c):
        return lax.div(a, jnp.full_like(a, c))

    days = div(t, 86400)
    z = days + 719468
    era = div(z, 146097)
    doe = z - era * 146097
    yoe = div(doe - div(doe, 1460) + div(doe, 36524) - div(doe, 146096), 365)
    y = yoe + era * 400
    doy = doe - (365 * yoe + div(yoe, 4) - div(yoe, 100))
    mp = div(5 * doy + 2, 153)
    d = doy - div(153 * mp + 2, 5) + 1
    m = jnp.where(mp < 10, mp + 3, mp - 9)
    y = jnp.where(m <= 2, y + 1, y)
    rel = jnp.minimum(jnp.maximum(y - BASE_YEAR, 0), NY - 2)
    combo = (rel + 1) * (NM * ND) + m * ND + d
    return jnp.where(t != 0, combo, t)


def _lookup_body(ts_hbm, table_hbm, out_hbm, ts_v, idx_v, rows_v, sem):
    wid = lax.axis_index("s") * 2 + lax.axis_index("c")
    base = wid * TOK_PER_W

    pltpu.sync_copy(ts_hbm.at[pl.ds(base, TOK_PER_W)], ts_v)

    def idx_body(j, _):
        off = j * 16
        idx_v[pl.ds(off, 16)] = _combo_from_ts(ts_v[pl.ds(off, 16)])
        return _

    lax.fori_loop(0, TOK_PER_W // 16, idx_body, None)

    def blk_body(i, _):
        off = i * GBLK
        pltpu.async_copy(
            table_hbm.at[idx_v.at[pl.ds(off, GBLK)]], rows_v, sem).wait()
        pltpu.sync_copy(rows_v, out_hbm.at[pl.ds(base + off, GBLK)])
        return _

    lax.fori_loop(0, NBLK, blk_body, None)


def _lookup(ts_flat, table):
    mesh = plsc.VectorSubcoreMesh(core_axis_name="c", subcore_axis_name="s")
    k = functools.partial(
        pl.kernel,
        mesh=mesh,
        out_type=jax.ShapeDtypeStruct((N_TOK, H), jnp.float32),
        scratch_types=[
            pltpu.VMEM((TOK_PER_W,), jnp.int32),
            pltpu.VMEM((TOK_PER_W,), jnp.int32),
            pltpu.VMEM((GBLK, H), jnp.float32),
            pltpu.SemaphoreType.DMA,
        ],
        compiler_params=pltpu.CompilerParams(use_tc_tiling_on_sc=False),
    )(_lookup_body)
    return k(ts_flat, table)


def kernel(ts_seq, year_table, month_table, day_table, W1, b1, W2, b2, gamma, beta):
    table = _build_table(year_table, month_table, day_table,
                         W1, b1, W2, b2, gamma, beta)
    out = _lookup(ts_seq.reshape(-1), table)
    return out.reshape(B, L, H)


# speedup vs baseline: 5.5648x; 1.1163x over previous
import functools

import jax
import jax.numpy as jnp
from jax import lax
from jax.experimental import pallas as pl
from jax.experimental.pallas import tpu as pltpu
from jax.experimental.pallas import tpu_sc as plsc

B, L, H = 4096, 50, 32
MAX_YEARS = 50
BASE_YEAR = 2000
NY, NM, ND = MAX_YEARS + 1, 13, 32
NCOMBO = NY * NM * ND
N_TOK = B * L

NW = 32
TOK_PER_W = N_TOK // NW
GBLK = 128
NBLK = TOK_PER_W // GBLK


def _table_body(yt, mt, dt, w1, b1, w2, b2, gamma, beta, t_out):
    f32 = jnp.float32
    yp = jnp.dot(yt[...], w1[0:H, :], preferred_element_type=f32)
    mp = jnp.dot(mt[...], w1[H:2 * H, :], preferred_element_type=f32)
    dp = jnp.dot(dt[...], w1[2 * H:3 * H, :], preferred_element_type=f32)

    r = lax.broadcasted_iota(jnp.int32, (NCOMBO, 1), 0)
    yi = r // (NM * ND)
    mi = (r // ND) % NM
    di = r % ND
    oh_y = (yi == lax.broadcasted_iota(jnp.int32, (NCOMBO, NY), 1)).astype(f32)
    oh_m = (mi == lax.broadcasted_iota(jnp.int32, (NCOMBO, NM), 1)).astype(f32)
    oh_d = (di == lax.broadcasted_iota(jnp.int32, (NCOMBO, ND), 1)).astype(f32)

    s = (jnp.dot(oh_y, yp, preferred_element_type=f32)
         + jnp.dot(oh_m, mp, preferred_element_type=f32)
         + jnp.dot(oh_d, dp, preferred_element_type=f32)
         + b1[...])
    h = jnp.maximum(s, 0.0)
    o = jnp.dot(h, w2[...], preferred_element_type=f32) + b2[...]
    mu = jnp.mean(o, axis=-1, keepdims=True)
    var = jnp.mean(jnp.square(o - mu), axis=-1, keepdims=True)
    normed = (o - mu) * lax.rsqrt(var + 1e-5)
    t_out[...] = normed * gamma[...] + beta[...]


def _build_table(year_table, month_table, day_table, W1, b1, W2, b2, gamma, beta):
    return pl.pallas_call(
        _table_body,
        out_shape=jax.ShapeDtypeStruct((NCOMBO, H), jnp.float32),
    )(year_table, month_table, day_table, W1,
      b1.reshape(1, H), W2, b2.reshape(1, H),
      gamma.reshape(1, H), beta.reshape(1, H))


def _combo_from_ts(t):
    i32 = jnp.int32

    def div(a, c):
        return lax.div(a, jnp.full_like(a, c))

    days = div(t, 86400)
    z = days + 719468
    era = div(z, 146097)
    doe = z - era * 146097
    yoe = div(doe - div(doe, 1460) + div(doe, 36524) - div(doe, 146096), 365)
    y = yoe + era * 400
    doy = doe - (365 * yoe + div(yoe, 4) - div(yoe, 100))
    mp = div(5 * doy + 2, 153)
    d = doy - div(153 * mp + 2, 5) + 1
    m = jnp.where(mp < 10, mp + 3, mp - 9)
    y = jnp.where(m <= 2, y + 1, y)
    rel = jnp.minimum(jnp.maximum(y - BASE_YEAR, 0), NY - 2)
    combo = (rel + 1) * (NM * ND) + m * ND + d
    return jnp.where(t != 0, combo, t)


CH = 1280
KG = CH // GBLK
NCH = TOK_PER_W // CH


def _lookup_body(ts_hbm, table_hbm, out_hbm,
                 ts_v, idx_v, rows0, rows1, gsem0, gsem1, ssem0, ssem1):
    wid = lax.axis_index("s") * 2 + lax.axis_index("c")
    base = wid * TOK_PER_W
    outf = out_hbm

    pltpu.sync_copy(ts_hbm.at[pl.ds(base, TOK_PER_W)], ts_v)

    rows = (rows0, rows1)
    gsems = (gsem0, gsem1)
    ssems = (ssem0, ssem1)

    def compute_idx(c):
        def body(j, _):
            off = c * CH + j * 16
            idx_v[pl.ds(off, 16)] = _combo_from_ts(ts_v[pl.ds(off, 16)])
            return _
        lax.fori_loop(0, CH // 16, body, None)

    def fire(c):
        buf, sem = rows[c % 2], gsems[c % 2]
        return [pltpu.async_copy(
                    table_hbm.at[idx_v.at[pl.ds(c * CH + j * GBLK, GBLK)]],
                    buf.at[pl.ds(j * GBLK, GBLK)], sem)
                for j in range(KG)]

    compute_idx(0)
    g = fire(0)
    store_h = [None, None]
    for c in range(NCH):
        if c + 1 < NCH:
            compute_idx(c + 1)
            if store_h[(c + 1) % 2] is not None:
                store_h[(c + 1) % 2].wait()
            g_next = fire(c + 1)
        for h in g:
            h.wait()
        store_h[c % 2] = pltpu.async_copy(
            rows[c % 2], outf.at[pl.ds(base + c * CH, CH)], ssems[c % 2])
        if c + 1 < NCH:
            g = g_next
    store_h[0].wait()
    store_h[1].wait()


def _lookup(ts_seq, table):
    mesh = plsc.VectorSubcoreMesh(core_axis_name="c", subcore_axis_name="s")
    k = functools.partial(
        pl.kernel,
        mesh=mesh,
        out_type=jax.ShapeDtypeStruct((N_TOK, H), jnp.float32),
        scratch_types=[
            pltpu.VMEM((TOK_PER_W,), jnp.int32),
            pltpu.VMEM((TOK_PER_W,), jnp.int32),
            pltpu.VMEM((CH, H), jnp.float32),
            pltpu.VMEM((CH, H), jnp.float32),
            pltpu.SemaphoreType.DMA,
            pltpu.SemaphoreType.DMA,
            pltpu.SemaphoreType.DMA,
            pltpu.SemaphoreType.DMA,
        ],
        compiler_params=pltpu.CompilerParams(use_tc_tiling_on_sc=False),
    )(_lookup_body)
    return k(ts_seq, table)


def kernel(ts_seq, year_table, month_table, day_table, W1, b1, W2, b2, gamma, beta):
    table = _build_table(year_table, month_table, day_table,
                         W1, b1, W2, b2, gamma, beta)
    return _lookup(ts_seq.reshape(-1), table).reshape(B, L, H)
